# Initial kernel scaffold; baseline (speedup 1.0000x reference)
import jax
import jax.numpy as jnp
from jax.experimental import pallas as pl


def kernel(features, channels, weights, noise_mean_full):
    raise NotImplementedError("write your pallas kernel here")


# speedup vs baseline: 37.9012x; 37.9012x over previous
import functools

import jax
import jax.numpy as jnp
from jax import lax
from jax.experimental import pallas as pl
from jax.experimental.pallas import tpu as pltpu
from jax.experimental.pallas import tpu_sc as plsc

N, RANK, CSUB, CFULL = 4096, 8, 16, 384
PSEUDO = 10.0

NC, NS, L = 2, 16, 16
NW = NC * NS
SPW = N // NW
ACC = RANK * CFULL
RED = ACC // NS


def _scatter_accumulate(feat_hbm, ch_hbm, w_hbm, out_hbm,
                        feat_v, ch_v, w_v, acc_v, red_v, tmp_v, shared):
    cid = lax.axis_index("c")
    sid = lax.axis_index("s")
    wid = cid * NS + sid
    base = wid * SPW

    pltpu.sync_copy(feat_hbm.at[pl.ds(base * RANK * CSUB, SPW * RANK * CSUB)], feat_v)
    pltpu.sync_copy(ch_hbm.at[pl.ds(base * CSUB, SPW * CSUB)], ch_v)
    pltpu.sync_copy(w_hbm.at[pl.ds(base, SPW)], w_v)

    def zero_body(i, _):
        acc_v[pl.ds(pl.multiple_of(i * L, L), L)] = jnp.zeros((L,), jnp.float32)
        return 0
    lax.fori_loop(0, ACC // L, zero_body, 0)

    iota = lax.iota(jnp.int32, L)
    shift = jnp.minimum(iota + 1, L - 1)
    last_lane = iota == L - 1

    def spike_body(n, _):
        ch = ch_v[pl.ds(pl.multiple_of(n * CSUB, CSUB), CSUB)]
        ch_next = ch.at[shift].get(mode="promise_in_bounds")
        keep = (ch != ch_next) | last_lane
        wg = w_v[pl.ds(pl.multiple_of((n // L) * L, L), L)]
        w = wg.at[jnp.full((L,), n % L, jnp.int32)].get(mode="promise_in_bounds")
        for r in range(RANK):
            row = feat_v[pl.ds(pl.multiple_of((n * RANK + r) * CSUB, CSUB), CSUB)]
            plsc.addupdate_scatter(acc_v, [ch + r * CFULL], row * w, mask=keep)
        return 0
    lax.fori_loop(0, SPW, spike_body, 0)

    pltpu.sync_copy(acc_v, shared.at[pl.ds(pl.multiple_of(sid * ACC, ACC), ACC)])
    plsc.subcore_barrier()

    off = pl.multiple_of(sid * RED, RED)
    pltpu.sync_copy(shared.at[pl.ds(off, RED)], red_v)
    for s in range(1, NS):
        pltpu.sync_copy(shared.at[pl.ds(s * ACC + off, RED)], tmp_v)
        for k in range(RED // L):
            sl = pl.ds(k * L, L)
            red_v[sl] = red_v[sl] + tmp_v[sl]

    pltpu.sync_copy(red_v, out_hbm.at[pl.ds(cid * ACC + off, RED)])


@functools.partial(
    pl.kernel,
    out_type=jax.ShapeDtypeStruct((NC * ACC,), jnp.float32),
    mesh=plsc.VectorSubcoreMesh(core_axis_name="c", subcore_axis_name="s"),
    compiler_params=pltpu.CompilerParams(needs_layout_passes=False),
    scratch_types=[
        pltpu.VMEM((SPW * RANK * CSUB,), jnp.float32),
        pltpu.VMEM((SPW * CSUB,), jnp.int32),
        pltpu.VMEM((SPW,), jnp.float32),
        pltpu.VMEM((ACC,), jnp.float32),
        pltpu.VMEM((RED,), jnp.float32),
        pltpu.VMEM((RED,), jnp.float32),
        pltpu.VMEM_SHARED((NS * ACC,), jnp.float32),
    ],
)
def _sc_partials(feat_hbm, ch_hbm, w_hbm, out_hbm,
                 feat_v, ch_v, w_v, acc_v, red_v, tmp_v, shared):
    _scatter_accumulate(feat_hbm, ch_hbm, w_hbm, out_hbm,
                        feat_v, ch_v, w_v, acc_v, red_v, tmp_v, shared)


def _finish_body(p_ref, w_ref, nm_ref, o_ref):
    total_w = jnp.sum(w_ref[...])
    s = p_ref[0] + p_ref[1]
    o_ref[...] = (s + PSEUDO * nm_ref[...]) * (1.0 / (PSEUDO + total_w))


def kernel(features, channels, weights, noise_mean_full):
    feat_flat = features.reshape(-1)
    ch_flat = channels.astype(jnp.int32).reshape(-1)
    partials = _sc_partials(feat_flat, ch_flat, weights)
    partials = partials.reshape(NC, RANK, CFULL)
    out = pl.pallas_call(
        _finish_body,
        out_shape=jax.ShapeDtypeStruct((RANK, CFULL), jnp.float32),
    )(partials, weights.reshape(NS * NC, SPW), noise_mean_full)
    return out


# speedup vs baseline: 41.0846x; 1.0840x over previous
import functools

import jax
import jax.numpy as jnp
from jax import lax
from jax.experimental import pallas as pl
from jax.experimental.pallas import tpu as pltpu
from jax.experimental.pallas import tpu_sc as plsc

N, RANK, CSUB, CFULL = 4096, 8, 16, 384
PSEUDO = 10.0

NC, NS, L = 2, 16, 16
NW = NC * NS
SPW = N // NW
ACC = RANK * CFULL
RED = ACC // NS


def _scatter_accumulate(feat_hbm, ch_hbm, w_hbm, out_hbm,
                        feat_v, ch_v, w_v, acc_v, red_v, tmp_v, shared):
    cid = lax.axis_index("c")
    sid = lax.axis_index("s")
    wid = cid * NS + sid
    base = wid * SPW

    pltpu.sync_copy(feat_hbm.at[pl.ds(base * RANK * CSUB, SPW * RANK * CSUB)], feat_v)
    pltpu.sync_copy(ch_hbm.at[pl.ds(base * CSUB, SPW * CSUB)], ch_v)
    pltpu.sync_copy(w_hbm.at[pl.ds(base, SPW)], w_v)

    def zero_body(i, _):
        acc_v[pl.ds(pl.multiple_of(i * L, L), L)] = jnp.zeros((L,), jnp.float32)
        return 0
    lax.fori_loop(0, ACC // L, zero_body, 0)

    iota = lax.iota(jnp.int32, L)
    shift = jnp.minimum(iota + 1, L - 1)
    last_lane = iota == L - 1

    @plsc.parallel_loop(0, SPW, unroll=4)
    def spike_body(n):
        ch = ch_v[pl.ds(pl.multiple_of(n * CSUB, CSUB), CSUB)]
        ch_next = ch.at[shift].get(mode="promise_in_bounds")
        keep = (ch != ch_next) | last_lane
        wg = w_v[pl.ds(pl.multiple_of((n // L) * L, L), L)]
        w = wg.at[jnp.full((L,), n % L, jnp.int32)].get(mode="promise_in_bounds")
        idx = ch
        for r in range(RANK):
            row = feat_v[pl.ds(pl.multiple_of((n * RANK + r) * CSUB, CSUB), CSUB)]
            plsc.addupdate_scatter(acc_v, [idx], row * w, mask=keep)
            if r < RANK - 1:
                idx = idx + CFULL

    pltpu.sync_copy(acc_v, shared.at[pl.ds(pl.multiple_of(sid * ACC, ACC), ACC)])
    plsc.subcore_barrier()

    off = pl.multiple_of(sid * RED, RED)
    pltpu.sync_copy(shared.at[pl.ds(off, RED)], red_v)

    def red_body(s, _):
        pltpu.sync_copy(shared.at[pl.ds(pl.multiple_of(s * ACC, ACC) + off, RED)], tmp_v)
        for k in range(RED // L):
            sl = pl.ds(k * L, L)
            red_v[sl] = red_v[sl] + tmp_v[sl]
        return 0
    lax.fori_loop(1, NS, red_body, 0)

    pltpu.sync_copy(red_v, out_hbm.at[pl.ds(cid * ACC + off, RED)])


@functools.partial(
    pl.kernel,
    out_type=jax.ShapeDtypeStruct((NC * ACC,), jnp.float32),
    mesh=plsc.VectorSubcoreMesh(core_axis_name="c", subcore_axis_name="s"),
    compiler_params=pltpu.CompilerParams(needs_layout_passes=False),
    scratch_types=[
        pltpu.VMEM((SPW * RANK * CSUB,), jnp.float32),
        pltpu.VMEM((SPW * CSUB,), jnp.int32),
        pltpu.VMEM((SPW,), jnp.float32),
        pltpu.VMEM((ACC,), jnp.float32),
        pltpu.VMEM((RED,), jnp.float32),
        pltpu.VMEM((RED,), jnp.float32),
        pltpu.VMEM_SHARED((NS * ACC,), jnp.float32),
    ],
)
def _sc_partials(feat_hbm, ch_hbm, w_hbm, out_hbm,
                 feat_v, ch_v, w_v, acc_v, red_v, tmp_v, shared):
    _scatter_accumulate(feat_hbm, ch_hbm, w_hbm, out_hbm,
                        feat_v, ch_v, w_v, acc_v, red_v, tmp_v, shared)


def _finish_body(p_ref, w_ref, nm_ref, o_ref):
    total_w = jnp.sum(w_ref[...])
    s = p_ref[0] + p_ref[1]
    o_ref[...] = (s + PSEUDO * nm_ref[...]) * (1.0 / (PSEUDO + total_w))


def kernel(features, channels, weights, noise_mean_full):
    feat_flat = features.reshape(-1)
    ch_flat = channels.astype(jnp.int32).reshape(-1)
    partials = _sc_partials(feat_flat, ch_flat, weights)
    partials = partials.reshape(NC, RANK, CFULL)
    out = pl.pallas_call(
        _finish_body,
        out_shape=jax.ShapeDtypeStruct((RANK, CFULL), jnp.float32),
    )(partials, weights.reshape(NS * NC, SPW), noise_mean_full)
    return out
